# baseline (speedup 1.0000x reference)
import functools
import math

import jax
import jax.numpy as jnp
import numpy as np
from jax import lax
from jax.experimental import pallas as pl
from jax.experimental.pallas import tpu as pltpu
from jax.experimental.pallas import tpu_sc as plsc

_N = 10000
_E = 320000
_NFEAT = 128
_NLABEL = 10
_NHID = 64

_NC = 2
_NS = 16
_W = _NC * _NS
_CHUNK = 128
_EW = _E // _W
_NT = _EW // _CHUNK
_TAIL = _EW - _NT * _CHUNK
_DEGW = 16

_BLK = 1000


def _make_seg_sum(with_deg):
    mesh = plsc.VectorSubcoreMesh(core_axis_name="c", subcore_axis_name="s")
    out_type = [jax.ShapeDtypeStruct((_NC, _N, _NHID), jnp.float32)]
    scratch = [
        pltpu.VMEM((_EW,), jnp.int32),
        pltpu.VMEM((_EW,), jnp.int32),
        pltpu.VMEM((_CHUNK, _NHID), jnp.float32),
        pltpu.VMEM((_CHUNK, _NHID), jnp.float32),
        pltpu.VMEM_SHARED((_N, _NHID), jnp.float32),
        pltpu.VMEM_SHARED((_N, _NHID), jnp.float32),
        pltpu.SemaphoreType.DMA,
        pltpu.SemaphoreType.DMA,
        pltpu.SemaphoreType.DMA,
        pltpu.SemaphoreType.DMA,
    ]
    if with_deg:
        out_type.append(jax.ShapeDtypeStruct((_NC, _N, _DEGW), jnp.float32))
        scratch += [
            pltpu.VMEM((_CHUNK, _DEGW), jnp.float32),
            pltpu.VMEM_SHARED((_N, _DEGW), jnp.float32),
        ]

    params = pltpu.CompilerParams(use_tc_tiling_on_sc=False)

    _ZCH = ((0, 128), (128, 128), (256, 128), (384, 128), (512, 113))

    def build(table, adj, out, degout, src_v, dst_v,
              rows_a, rows_b, table_sh, acc_sh, g_a, g_b, s_a, s_b,
              ones_v, deg_sh):
        s = lax.axis_index("s")
        c = lax.axis_index("c")
        wid = s * _NC + c
        zr = _N // _NS
        pltpu.sync_copy(table.at[pl.ds(s * zr, zr)],
                        table_sh.at[pl.ds(s * zr, zr)])
        zv = jnp.zeros((16,), jnp.float32)
        for r in range(_CHUNK):
            for cb in range(_NHID // 16):
                rows_a[r, pl.ds(cb * 16, 16)] = zv
        base = s * zr
        for kk, sz in _ZCH:
            pltpu.sync_copy(rows_a.at[pl.ds(0, sz)],
                            acc_sh.at[pl.ds(base + kk, sz)])
        if with_deg:
            for r in range(_CHUNK):
                ones_v[r, pl.ds(0, _DEGW)] = zv
            for kk, sz in _ZCH:
                pltpu.sync_copy(ones_v.at[pl.ds(0, sz)],
                                deg_sh.at[pl.ds(base + kk, sz)])
            ov = jnp.ones((16,), jnp.float32)
            for r in range(_CHUNK):
                ones_v[r, pl.ds(0, _DEGW)] = ov
        pltpu.sync_copy(adj.at[0, pl.ds(wid * _EW, _EW)], src_v)
        pltpu.sync_copy(adj.at[1, pl.ds(wid * _EW, _EW)], dst_v)
        plsc.subcore_barrier()

        def fire_gather(buf, i, sem):
            pltpu.async_copy(
                table_sh.at[src_v.at[pl.ds(i * _CHUNK, _CHUNK)]], buf, sem)

        def wait_gather(buf, sem):
            pltpu.make_async_copy(table.at[pl.ds(0, _CHUNK)], buf, sem).wait()

        def fire_scatter(buf, i, sem):
            idx = dst_v.at[pl.ds(i * _CHUNK, _CHUNK)]
            pltpu.async_copy(buf, acc_sh.at[idx], sem, add=True)
            if with_deg:
                pltpu.async_copy(ones_v, deg_sh.at[idx], sem, add=True)

        def wait_scatter(buf, sem):
            pltpu.make_async_copy(buf, acc_sh.at[pl.ds(0, _CHUNK)], sem).wait()
            if with_deg:
                pltpu.make_async_copy(ones_v, deg_sh.at[pl.ds(0, _CHUNK)],
                                      sem).wait()

        fire_gather(rows_a, 0, g_a)

        def body2(k, carry):
            i0 = k * 2
            for (off, cur, curg, curs, oth, othg, oths) in (
                    (0, rows_a, g_a, s_a, rows_b, g_b, s_b),
                    (1, rows_b, g_b, s_b, rows_a, g_a, s_a)):
                i = i0 + off
                wait_gather(cur, curg)
                fire_scatter(cur, i, curs)

                @pl.when(i >= 1)
                def _():
                    wait_scatter(oth, oths)

                @pl.when(i + 1 < _NT)
                def _():
                    fire_gather(oth, i + 1, othg)
            return carry

        lax.fori_loop(0, _NT // 2, body2, 0)
        wait_scatter(rows_b, s_b)

        tb = _NT * _CHUNK
        pltpu.async_copy(table_sh.at[src_v.at[pl.ds(tb, _TAIL)]],
                         rows_a.at[pl.ds(0, _TAIL)], g_a).wait()
        pltpu.sync_copy(rows_a.at[pl.ds(0, _TAIL)],
                        acc_sh.at[dst_v.at[pl.ds(tb, _TAIL)]], add=True)
        if with_deg:
            pltpu.sync_copy(ones_v.at[pl.ds(0, _TAIL)],
                            deg_sh.at[dst_v.at[pl.ds(tb, _TAIL)]], add=True)

        plsc.subcore_barrier()
        orow = _N // _NS
        pltpu.sync_copy(acc_sh.at[pl.ds(s * orow, orow)],
                        out.at[c, pl.ds(s * orow, orow)])
        if with_deg:
            pltpu.sync_copy(deg_sh.at[pl.ds(s * orow, orow)],
                            degout.at[c, pl.ds(s * orow, orow)])

    if with_deg:
        @functools.partial(pl.kernel, mesh=mesh, out_type=out_type,
                           scratch_types=scratch, compiler_params=params)
        def seg(table, adj, out, degout, src_v, dst_v,
                rows_a, rows_b, table_sh, acc_sh, g_a, g_b, s_a, s_b,
                ones_v, deg_sh):
            build(table, adj, out, degout, src_v, dst_v,
                  rows_a, rows_b, table_sh, acc_sh, g_a, g_b, s_a, s_b,
                  ones_v, deg_sh)

        return seg

    @functools.partial(pl.kernel, mesh=mesh, out_type=out_type[0],
                       scratch_types=scratch, compiler_params=params)
    def seg(table, adj, out, src_v, dst_v,
            rows_a, rows_b, table_sh, acc_sh, g_a, g_b, s_a, s_b):
        build(table, adj, out, None, src_v, dst_v,
              rows_a, rows_b, table_sh, acc_sh, g_a, g_b, s_a, s_b,
              None, None)

    return seg


_seg_sum_deg = _make_seg_sum(True)
_seg_sum = _make_seg_sum(False)


def _elu(v):
    return jnp.where(v > 0, v, jnp.exp(v) - 1.0)


def _row_spec(width):
    return pl.BlockSpec((_BLK, width), lambda i: (i, 0))


def _full_spec(shape):
    nd = len(shape)
    return pl.BlockSpec(shape, lambda i: (0,) * nd)


def _acc_spec(width):
    return pl.BlockSpec((_NC, _BLK, width), lambda i: (0, i, 0))


def _dense0_body(t_ref, x_ref, q_ref, fr_ref, wt1, bt1, wt2, bt2,
                 wl0x, wl0q, wr0x, wr0q, temb_ref, p0_ref, r0_ref):
    tt = t_ref[...]
    ph = tt * fr_ref[...]
    lane = lax.broadcasted_iota(jnp.int32, ph.shape, 1)
    temb0 = jnp.where(lane < _NHID // 2, jnp.sin(ph), jnp.cos(ph))
    hmid = _elu(jnp.dot(temb0, wt1[...], preferred_element_type=jnp.float32)
                + bt1[...])
    temb = jnp.dot(hmid, wt2[...], preferred_element_type=jnp.float32) + bt2[...]
    temb_ref[...] = temb
    x = x_ref[...]
    q = q_ref[...]
    p0_ref[...] = (jnp.dot(x, wl0x[...], preferred_element_type=jnp.float32)
                   + jnp.dot(q, wl0q[...], preferred_element_type=jnp.float32))
    r0_ref[...] = (jnp.dot(x, wr0x[...], preferred_element_type=jnp.float32)
                   + jnp.dot(q, wr0q[...], preferred_element_type=jnp.float32))


def _dense0(t2, x, q, fr, wt1, bt1, wt2, bt2, wl0x, wl0q, wr0x, wr0q):
    grid = (_N // _BLK,)
    return pl.pallas_call(
        _dense0_body,
        grid=grid,
        in_specs=[
            _row_spec(1), _row_spec(_NFEAT), _row_spec(_NLABEL),
            _full_spec(fr.shape), _full_spec(wt1.shape), _full_spec(bt1.shape),
            _full_spec(wt2.shape), _full_spec(bt2.shape),
            _full_spec(wl0x.shape), _full_spec(wl0q.shape),
            _full_spec(wr0x.shape), _full_spec(wr0q.shape),
        ],
        out_specs=[_row_spec(_NHID), _row_spec(_NHID), _row_spec(_NHID)],
        out_shape=[jax.ShapeDtypeStruct((_N, _NHID), jnp.float32)] * 3,
    )(t2, x, q, fr, wt1, bt1, wt2, bt2, wl0x, wl0q, wr0x, wr0q)


def _sage_post(acc, dg, r, bl, temb):
    agg = acc[0] + acc[1]
    deg = dg[0, :, 0:1] + dg[1, :, 0:1]
    out = agg / jnp.maximum(deg, 1.0) + bl[...] + r[...]
    nrm = jnp.sqrt(jnp.sum(out * out, axis=1, keepdims=True))
    out = out / jnp.maximum(nrm, 1e-12)
    return jnp.maximum(out + temb[...], 0.0)


def _dense1_body(acc, dg, r0, temb, q_ref, bl0,
                 wl1h, wl1q, wr1h, wr1q, p1_ref, r1_ref):
    h = _sage_post(acc, dg, r0, bl0, temb)
    q = q_ref[...]
    p1_ref[...] = (jnp.dot(h, wl1h[...], preferred_element_type=jnp.float32)
                   + jnp.dot(q, wl1q[...], preferred_element_type=jnp.float32))
    r1_ref[...] = (jnp.dot(h, wr1h[...], preferred_element_type=jnp.float32)
                   + jnp.dot(q, wr1q[...], preferred_element_type=jnp.float32))


def _dense1(acc, dg, r0, temb, q, bl0, wl1h, wl1q, wr1h, wr1q):
    grid = (_N // _BLK,)
    return pl.pallas_call(
        _dense1_body,
        grid=grid,
        in_specs=[
            _acc_spec(_NHID), _acc_spec(_DEGW),
            _row_spec(_NHID), _row_spec(_NHID), _row_spec(_NLABEL),
            _full_spec(bl0.shape),
            _full_spec(wl1h.shape), _full_spec(wl1q.shape),
            _full_spec(wr1h.shape), _full_spec(wr1q.shape),
        ],
        out_specs=[_row_spec(_NHID), _row_spec(_NHID)],
        out_shape=[jax.ShapeDtypeStruct((_N, _NHID), jnp.float32)] * 2,
    )(acc, dg, r0, temb, q, bl0, wl1h, wl1q, wr1h, wr1q)


def _dense2_body(acc, dg, r1, temb, q_ref, bl1,
                 wf1h, wf1q, bf1, wf2, bf2, out_ref):
    h = _sage_post(acc, dg, r1, bl1, temb)
    q = q_ref[...]
    f = _elu(jnp.dot(h, wf1h[...], preferred_element_type=jnp.float32)
             + jnp.dot(q, wf1q[...], preferred_element_type=jnp.float32)
             + bf1[...])
    out_ref[...] = jnp.dot(f, wf2[...], preferred_element_type=jnp.float32) + bf2[...]


def _dense2(acc, dg, r1, temb, q, bl1, wf1h, wf1q, bf1, wf2, bf2):
    grid = (_N // _BLK,)
    return pl.pallas_call(
        _dense2_body,
        grid=grid,
        in_specs=[
            _acc_spec(_NHID), _acc_spec(_DEGW),
            _row_spec(_NHID), _row_spec(_NHID), _row_spec(_NLABEL),
            _full_spec(bl1.shape),
            _full_spec(wf1h.shape), _full_spec(wf1q.shape),
            _full_spec(bf1.shape), _full_spec(wf2.shape), _full_spec(bf2.shape),
        ],
        out_specs=[_row_spec(_NLABEL)],
        out_shape=[jax.ShapeDtypeStruct((_N, _NLABEL), jnp.float32)],
    )(acc, dg, r1, temb, q, bl1, wf1h, wf1q, bf1, wf2, bf2)[0]


_FREQS = np.exp(np.arange(_NHID // 2, dtype=np.float64)
                * (-math.log(10000.0) / (_NHID // 2 - 1))).astype(np.float32)


def kernel(x, q_Y_sample, adj, t, num_steps, W_t1, b_t1, W_t2, b_t2,
           Wl0, bl0, Wr0, Wl1, bl1, Wr1, Wf1, bf1, Wf2, bf2):
    q = q_Y_sample
    adj32 = adj.astype(jnp.int32)

    t2 = (t / num_steps * num_steps * 4.0).reshape(_N, 1)
    fr = jnp.asarray(np.concatenate([_FREQS, _FREQS])[None, :])

    temb, p0, r0 = _dense0(
        t2, x, q, fr, W_t1, b_t1.reshape(1, -1), W_t2, b_t2.reshape(1, -1),
        Wl0[:_NFEAT], Wl0[_NFEAT:], Wr0[:_NFEAT], Wr0[_NFEAT:])

    acc0, deg0 = _seg_sum_deg(p0, adj32)

    p1, r1 = _dense1(
        acc0, deg0, r0, temb, q, bl0.reshape(1, -1),
        Wl1[:_NHID], Wl1[_NHID:], Wr1[:_NHID], Wr1[_NHID:])

    acc1 = _seg_sum(p1, adj32)

    out = _dense2(
        acc1, deg0, r1, temb, q, bl1.reshape(1, -1),
        Wf1[:_NHID], Wf1[_NHID:], bf1.reshape(1, -1), Wf2, bf2.reshape(1, -1))
    return out


# speedup vs baseline: 1.0209x; 1.0209x over previous
import functools
import math

import jax
import jax.numpy as jnp
import numpy as np
from jax import lax
from jax.experimental import pallas as pl
from jax.experimental.pallas import tpu as pltpu
from jax.experimental.pallas import tpu_sc as plsc

_N = 10000
_E = 320000
_NFEAT = 128
_NLABEL = 10
_NHID = 64

_NC = 2
_NS = 16
_W = _NC * _NS
_CHUNK = 128
_EW = _E // _W
_NT = _EW // _CHUNK
_TAIL = _EW - _NT * _CHUNK
_DEGW = 16

_BLK = 5000


def _make_seg_sum(with_deg):
    mesh = plsc.VectorSubcoreMesh(core_axis_name="c", subcore_axis_name="s")
    out_type = [jax.ShapeDtypeStruct((_NC, _N, _NHID), jnp.float32)]
    scratch = [
        pltpu.VMEM((_EW,), jnp.int32),
        pltpu.VMEM((_EW,), jnp.int32),
        pltpu.VMEM((_CHUNK, _NHID), jnp.float32),
        pltpu.VMEM((_CHUNK, _NHID), jnp.float32),
        pltpu.VMEM_SHARED((_N, _NHID), jnp.float32),
        pltpu.VMEM_SHARED((_N, _NHID), jnp.float32),
        pltpu.SemaphoreType.DMA,
        pltpu.SemaphoreType.DMA,
        pltpu.SemaphoreType.DMA,
        pltpu.SemaphoreType.DMA,
    ]
    if with_deg:
        out_type.append(jax.ShapeDtypeStruct((_NC, _N, _DEGW), jnp.float32))
        scratch += [
            pltpu.VMEM((_CHUNK, _DEGW), jnp.float32),
            pltpu.VMEM_SHARED((_N, _DEGW), jnp.float32),
        ]

    params = pltpu.CompilerParams(use_tc_tiling_on_sc=False)

    _ZCH = ((0, 128), (128, 128), (256, 128), (384, 128), (512, 113))

    def build(table, adj, out, degout, src_v, dst_v,
              rows_a, rows_b, table_sh, acc_sh, g_a, g_b, s_a, s_b,
              ones_v, deg_sh):
        s = lax.axis_index("s")
        c = lax.axis_index("c")
        wid = s * _NC + c
        zr = _N // _NS
        pltpu.sync_copy(table.at[pl.ds(s * zr, zr)],
                        table_sh.at[pl.ds(s * zr, zr)])
        zv = jnp.zeros((16,), jnp.float32)
        for r in range(_CHUNK):
            for cb in range(_NHID // 16):
                rows_a[r, pl.ds(cb * 16, 16)] = zv
        base = s * zr
        for kk, sz in _ZCH:
            pltpu.sync_copy(rows_a.at[pl.ds(0, sz)],
                            acc_sh.at[pl.ds(base + kk, sz)])
        if with_deg:
            for r in range(_CHUNK):
                ones_v[r, pl.ds(0, _DEGW)] = zv
            for kk, sz in _ZCH:
                pltpu.sync_copy(ones_v.at[pl.ds(0, sz)],
                                deg_sh.at[pl.ds(base + kk, sz)])
            ov = jnp.ones((16,), jnp.float32)
            for r in range(_CHUNK):
                ones_v[r, pl.ds(0, _DEGW)] = ov
        pltpu.sync_copy(adj.at[0, pl.ds(wid * _EW, _EW)], src_v)
        pltpu.sync_copy(adj.at[1, pl.ds(wid * _EW, _EW)], dst_v)
        plsc.subcore_barrier()

        def fire_gather(buf, i, sem):
            pltpu.async_copy(
                table_sh.at[src_v.at[pl.ds(i * _CHUNK, _CHUNK)]], buf, sem)

        def wait_gather(buf, sem):
            pltpu.make_async_copy(table.at[pl.ds(0, _CHUNK)], buf, sem).wait()

        def fire_scatter(buf, i, sem):
            idx = dst_v.at[pl.ds(i * _CHUNK, _CHUNK)]
            pltpu.async_copy(buf, acc_sh.at[idx], sem, add=True)
            if with_deg:
                pltpu.async_copy(ones_v, deg_sh.at[idx], sem, add=True)

        def wait_scatter(buf, sem):
            pltpu.make_async_copy(buf, acc_sh.at[pl.ds(0, _CHUNK)], sem).wait()
            if with_deg:
                pltpu.make_async_copy(ones_v, deg_sh.at[pl.ds(0, _CHUNK)],
                                      sem).wait()

        fire_gather(rows_a, 0, g_a)

        def body2(k, carry):
            i0 = k * 2
            for (off, cur, curg, curs, oth, othg, oths) in (
                    (0, rows_a, g_a, s_a, rows_b, g_b, s_b),
                    (1, rows_b, g_b, s_b, rows_a, g_a, s_a)):
                i = i0 + off
                wait_gather(cur, curg)
                fire_scatter(cur, i, curs)

                @pl.when(i >= 1)
                def _():
                    wait_scatter(oth, oths)

                @pl.when(i + 1 < _NT)
                def _():
                    fire_gather(oth, i + 1, othg)
            return carry

        lax.fori_loop(0, _NT // 2, body2, 0)
        wait_scatter(rows_b, s_b)

        tb = _NT * _CHUNK
        pltpu.async_copy(table_sh.at[src_v.at[pl.ds(tb, _TAIL)]],
                         rows_a.at[pl.ds(0, _TAIL)], g_a).wait()
        pltpu.sync_copy(rows_a.at[pl.ds(0, _TAIL)],
                        acc_sh.at[dst_v.at[pl.ds(tb, _TAIL)]], add=True)
        if with_deg:
            pltpu.sync_copy(ones_v.at[pl.ds(0, _TAIL)],
                            deg_sh.at[dst_v.at[pl.ds(tb, _TAIL)]], add=True)

        plsc.subcore_barrier()
        orow = _N // _NS
        pltpu.sync_copy(acc_sh.at[pl.ds(s * orow, orow)],
                        out.at[c, pl.ds(s * orow, orow)])
        if with_deg:
            pltpu.sync_copy(deg_sh.at[pl.ds(s * orow, orow)],
                            degout.at[c, pl.ds(s * orow, orow)])

    if with_deg:
        @functools.partial(pl.kernel, mesh=mesh, out_type=out_type,
                           scratch_types=scratch, compiler_params=params)
        def seg(table, adj, out, degout, src_v, dst_v,
                rows_a, rows_b, table_sh, acc_sh, g_a, g_b, s_a, s_b,
                ones_v, deg_sh):
            build(table, adj, out, degout, src_v, dst_v,
                  rows_a, rows_b, table_sh, acc_sh, g_a, g_b, s_a, s_b,
                  ones_v, deg_sh)

        return seg

    @functools.partial(pl.kernel, mesh=mesh, out_type=out_type[0],
                       scratch_types=scratch, compiler_params=params)
    def seg(table, adj, out, src_v, dst_v,
            rows_a, rows_b, table_sh, acc_sh, g_a, g_b, s_a, s_b):
        build(table, adj, out, None, src_v, dst_v,
              rows_a, rows_b, table_sh, acc_sh, g_a, g_b, s_a, s_b,
              None, None)

    return seg


_seg_sum_deg = _make_seg_sum(True)
_seg_sum = _make_seg_sum(False)


def _elu(v):
    return jnp.where(v > 0, v, jnp.exp(v) - 1.0)


def _row_spec(width):
    return pl.BlockSpec((_BLK, width), lambda i: (i, 0))


def _full_spec(shape):
    nd = len(shape)
    return pl.BlockSpec(shape, lambda i: (0,) * nd)


def _acc_spec(width):
    return pl.BlockSpec((_NC, _BLK, width), lambda i: (0, i, 0))


def _dense0_body(t_ref, x_ref, q_ref, fr_ref, wt1, bt1, wt2, bt2,
                 wl0x, wl0q, wr0x, wr0q, temb_ref, p0_ref, r0_ref):
    tt = t_ref[...]
    ph = tt * fr_ref[...]
    lane = lax.broadcasted_iota(jnp.int32, ph.shape, 1)
    temb0 = jnp.where(lane < _NHID // 2, jnp.sin(ph), jnp.cos(ph))
    hmid = _elu(jnp.dot(temb0, wt1[...], preferred_element_type=jnp.float32)
                + bt1[...])
    temb = jnp.dot(hmid, wt2[...], preferred_element_type=jnp.float32) + bt2[...]
    temb_ref[...] = temb
    x = x_ref[...]
    q = q_ref[...]
    p0_ref[...] = (jnp.dot(x, wl0x[...], preferred_element_type=jnp.float32)
                   + jnp.dot(q, wl0q[...], preferred_element_type=jnp.float32))
    r0_ref[...] = (jnp.dot(x, wr0x[...], preferred_element_type=jnp.float32)
                   + jnp.dot(q, wr0q[...], preferred_element_type=jnp.float32))


def _dense0(t2, x, q, fr, wt1, bt1, wt2, bt2, wl0x, wl0q, wr0x, wr0q):
    grid = (_N // _BLK,)
    return pl.pallas_call(
        _dense0_body,
        grid=grid,
        in_specs=[
            _row_spec(1), _row_spec(_NFEAT), _row_spec(_NLABEL),
            _full_spec(fr.shape), _full_spec(wt1.shape), _full_spec(bt1.shape),
            _full_spec(wt2.shape), _full_spec(bt2.shape),
            _full_spec(wl0x.shape), _full_spec(wl0q.shape),
            _full_spec(wr0x.shape), _full_spec(wr0q.shape),
        ],
        out_specs=[_row_spec(_NHID), _row_spec(_NHID), _row_spec(_NHID)],
        out_shape=[jax.ShapeDtypeStruct((_N, _NHID), jnp.float32)] * 3,
    )(t2, x, q, fr, wt1, bt1, wt2, bt2, wl0x, wl0q, wr0x, wr0q)


def _sage_post(acc, dg, r, bl, temb):
    agg = acc[0] + acc[1]
    deg = dg[0, :, 0:1] + dg[1, :, 0:1]
    out = agg / jnp.maximum(deg, 1.0) + bl[...] + r[...]
    nrm = jnp.sqrt(jnp.sum(out * out, axis=1, keepdims=True))
    out = out / jnp.maximum(nrm, 1e-12)
    return jnp.maximum(out + temb[...], 0.0)


def _dense1_body(acc, dg, r0, temb, q_ref, bl0,
                 wl1h, wl1q, wr1h, wr1q, p1_ref, r1_ref):
    h = _sage_post(acc, dg, r0, bl0, temb)
    q = q_ref[...]
    p1_ref[...] = (jnp.dot(h, wl1h[...], preferred_element_type=jnp.float32)
                   + jnp.dot(q, wl1q[...], preferred_element_type=jnp.float32))
    r1_ref[...] = (jnp.dot(h, wr1h[...], preferred_element_type=jnp.float32)
                   + jnp.dot(q, wr1q[...], preferred_element_type=jnp.float32))


def _dense1(acc, dg, r0, temb, q, bl0, wl1h, wl1q, wr1h, wr1q):
    grid = (_N // _BLK,)
    return pl.pallas_call(
        _dense1_body,
        grid=grid,
        in_specs=[
            _acc_spec(_NHID), _acc_spec(_DEGW),
            _row_spec(_NHID), _row_spec(_NHID), _row_spec(_NLABEL),
            _full_spec(bl0.shape),
            _full_spec(wl1h.shape), _full_spec(wl1q.shape),
            _full_spec(wr1h.shape), _full_spec(wr1q.shape),
        ],
        out_specs=[_row_spec(_NHID), _row_spec(_NHID)],
        out_shape=[jax.ShapeDtypeStruct((_N, _NHID), jnp.float32)] * 2,
    )(acc, dg, r0, temb, q, bl0, wl1h, wl1q, wr1h, wr1q)


def _dense2_body(acc, dg, r1, temb, q_ref, bl1,
                 wf1h, wf1q, bf1, wf2, bf2, out_ref):
    h = _sage_post(acc, dg, r1, bl1, temb)
    q = q_ref[...]
    f = _elu(jnp.dot(h, wf1h[...], preferred_element_type=jnp.float32)
             + jnp.dot(q, wf1q[...], preferred_element_type=jnp.float32)
             + bf1[...])
    out_ref[...] = jnp.dot(f, wf2[...], preferred_element_type=jnp.float32) + bf2[...]


def _dense2(acc, dg, r1, temb, q, bl1, wf1h, wf1q, bf1, wf2, bf2):
    grid = (_N // _BLK,)
    return pl.pallas_call(
        _dense2_body,
        grid=grid,
        in_specs=[
            _acc_spec(_NHID), _acc_spec(_DEGW),
            _row_spec(_NHID), _row_spec(_NHID), _row_spec(_NLABEL),
            _full_spec(bl1.shape),
            _full_spec(wf1h.shape), _full_spec(wf1q.shape),
            _full_spec(bf1.shape), _full_spec(wf2.shape), _full_spec(bf2.shape),
        ],
        out_specs=[_row_spec(_NLABEL)],
        out_shape=[jax.ShapeDtypeStruct((_N, _NLABEL), jnp.float32)],
    )(acc, dg, r1, temb, q, bl1, wf1h, wf1q, bf1, wf2, bf2)[0]


_FREQS = np.exp(np.arange(_NHID // 2, dtype=np.float64)
                * (-math.log(10000.0) / (_NHID // 2 - 1))).astype(np.float32)


def kernel(x, q_Y_sample, adj, t, num_steps, W_t1, b_t1, W_t2, b_t2,
           Wl0, bl0, Wr0, Wl1, bl1, Wr1, Wf1, bf1, Wf2, bf2):
    q = q_Y_sample
    adj32 = adj.astype(jnp.int32)

    t2 = (t / num_steps * num_steps * 4.0).reshape(_N, 1)
    fr = jnp.asarray(np.concatenate([_FREQS, _FREQS])[None, :])

    temb, p0, r0 = _dense0(
        t2, x, q, fr, W_t1, b_t1.reshape(1, -1), W_t2, b_t2.reshape(1, -1),
        Wl0[:_NFEAT], Wl0[_NFEAT:], Wr0[:_NFEAT], Wr0[_NFEAT:])

    acc0, deg0 = _seg_sum_deg(p0, adj32)

    p1, r1 = _dense1(
        acc0, deg0, r0, temb, q, bl0.reshape(1, -1),
        Wl1[:_NHID], Wl1[_NHID:], Wr1[:_NHID], Wr1[_NHID:])

    acc1 = _seg_sum(p1, adj32)

    out = _dense2(
        acc1, deg0, r1, temb, q, bl1.reshape(1, -1),
        Wf1[:_NHID], Wf1[_NHID:], bf1.reshape(1, -1), Wf2, bf2.reshape(1, -1))
    return out


# speedup vs baseline: 1.0308x; 1.0097x over previous
import functools
import math

import jax
import jax.numpy as jnp
import numpy as np
from jax import lax
from jax.experimental import pallas as pl
from jax.experimental.pallas import tpu as pltpu
from jax.experimental.pallas import tpu_sc as plsc

_N = 10000
_E = 320000
_NFEAT = 128
_NLABEL = 10
_NHID = 64

_NC = 2
_NS = 16
_W = _NC * _NS
_CHUNK = 128
_EW = _E // _W
_NT = _EW // _CHUNK
_TAIL = _EW - _NT * _CHUNK
_DEGW = 16

_BLK = 2000


def _make_seg_sum(with_deg):
    mesh = plsc.VectorSubcoreMesh(core_axis_name="c", subcore_axis_name="s")
    out_type = [jax.ShapeDtypeStruct((_NC, _N, _NHID), jnp.float32)]
    scratch = [
        pltpu.VMEM((_EW,), jnp.int32),
        pltpu.VMEM((_EW,), jnp.int32),
        pltpu.VMEM((_CHUNK, _NHID), jnp.float32),
        pltpu.VMEM((_CHUNK, _NHID), jnp.float32),
        pltpu.VMEM_SHARED((_N, _NHID), jnp.float32),
        pltpu.VMEM_SHARED((_N, _NHID), jnp.float32),
        pltpu.SemaphoreType.DMA,
        pltpu.SemaphoreType.DMA,
        pltpu.SemaphoreType.DMA,
        pltpu.SemaphoreType.DMA,
    ]
    if with_deg:
        out_type.append(jax.ShapeDtypeStruct((_NC, _N, _DEGW), jnp.float32))
        scratch += [
            pltpu.VMEM((_CHUNK, _DEGW), jnp.float32),
            pltpu.VMEM_SHARED((_N, _DEGW), jnp.float32),
        ]

    params = pltpu.CompilerParams(use_tc_tiling_on_sc=False)

    _ZCH = ((0, 128), (128, 128), (256, 128), (384, 128), (512, 113))

    def build(table, adj, out, degout, src_v, dst_v,
              rows_a, rows_b, table_sh, acc_sh, g_a, g_b, s_a, s_b,
              ones_v, deg_sh):
        s = lax.axis_index("s")
        c = lax.axis_index("c")
        wid = s * _NC + c
        zr = _N // _NS
        pltpu.sync_copy(table.at[pl.ds(s * zr, zr)],
                        table_sh.at[pl.ds(s * zr, zr)])
        zv = jnp.zeros((16,), jnp.float32)
        for r in range(_CHUNK):
            for cb in range(_NHID // 16):
                rows_a[r, pl.ds(cb * 16, 16)] = zv
        base = s * zr
        for kk, sz in _ZCH:
            pltpu.sync_copy(rows_a.at[pl.ds(0, sz)],
                            acc_sh.at[pl.ds(base + kk, sz)])
        if with_deg:
            for r in range(_CHUNK):
                ones_v[r, pl.ds(0, _DEGW)] = zv
            for kk, sz in _ZCH:
                pltpu.sync_copy(ones_v.at[pl.ds(0, sz)],
                                deg_sh.at[pl.ds(base + kk, sz)])
            ov = jnp.ones((16,), jnp.float32)
            for r in range(_CHUNK):
                ones_v[r, pl.ds(0, _DEGW)] = ov
        pltpu.sync_copy(adj.at[0, pl.ds(wid * _EW, _EW)], src_v)
        pltpu.sync_copy(adj.at[1, pl.ds(wid * _EW, _EW)], dst_v)
        plsc.subcore_barrier()

        def fire_gather(buf, i, sem):
            pltpu.async_copy(
                table_sh.at[src_v.at[pl.ds(i * _CHUNK, _CHUNK)]], buf, sem)

        def wait_gather(buf, sem):
            pltpu.make_async_copy(table.at[pl.ds(0, _CHUNK)], buf, sem).wait()

        def fire_scatter(buf, i, sem):
            idx = dst_v.at[pl.ds(i * _CHUNK, _CHUNK)]
            pltpu.async_copy(buf, acc_sh.at[idx], sem, add=True)
            if with_deg:
                pltpu.async_copy(ones_v, deg_sh.at[idx], sem, add=True)

        def wait_scatter(buf, sem):
            pltpu.make_async_copy(buf, acc_sh.at[pl.ds(0, _CHUNK)], sem).wait()
            if with_deg:
                pltpu.make_async_copy(ones_v, deg_sh.at[pl.ds(0, _CHUNK)],
                                      sem).wait()

        fire_gather(rows_a, 0, g_a)

        def body2(k, carry):
            i0 = k * 2
            for (off, cur, curg, curs, oth, othg, oths) in (
                    (0, rows_a, g_a, s_a, rows_b, g_b, s_b),
                    (1, rows_b, g_b, s_b, rows_a, g_a, s_a)):
                i = i0 + off
                wait_gather(cur, curg)
                fire_scatter(cur, i, curs)

                @pl.when(i >= 1)
                def _():
                    wait_scatter(oth, oths)

                @pl.when(i + 1 < _NT)
                def _():
                    fire_gather(oth, i + 1, othg)
            return carry

        lax.fori_loop(0, _NT // 2, body2, 0)
        wait_scatter(rows_b, s_b)

        tb = _NT * _CHUNK
        pltpu.async_copy(table_sh.at[src_v.at[pl.ds(tb, _TAIL)]],
                         rows_a.at[pl.ds(0, _TAIL)], g_a).wait()
        pltpu.sync_copy(rows_a.at[pl.ds(0, _TAIL)],
                        acc_sh.at[dst_v.at[pl.ds(tb, _TAIL)]], add=True)
        if with_deg:
            pltpu.sync_copy(ones_v.at[pl.ds(0, _TAIL)],
                            deg_sh.at[dst_v.at[pl.ds(tb, _TAIL)]], add=True)

        plsc.subcore_barrier()
        orow = _N // _NS
        pltpu.sync_copy(acc_sh.at[pl.ds(s * orow, orow)],
                        out.at[c, pl.ds(s * orow, orow)])
        if with_deg:
            pltpu.sync_copy(deg_sh.at[pl.ds(s * orow, orow)],
                            degout.at[c, pl.ds(s * orow, orow)])

    if with_deg:
        @functools.partial(pl.kernel, mesh=mesh, out_type=out_type,
                           scratch_types=scratch, compiler_params=params)
        def seg(table, adj, out, degout, src_v, dst_v,
                rows_a, rows_b, table_sh, acc_sh, g_a, g_b, s_a, s_b,
                ones_v, deg_sh):
            build(table, adj, out, degout, src_v, dst_v,
                  rows_a, rows_b, table_sh, acc_sh, g_a, g_b, s_a, s_b,
                  ones_v, deg_sh)

        return seg

    @functools.partial(pl.kernel, mesh=mesh, out_type=out_type[0],
                       scratch_types=scratch, compiler_params=params)
    def seg(table, adj, out, src_v, dst_v,
            rows_a, rows_b, table_sh, acc_sh, g_a, g_b, s_a, s_b):
        build(table, adj, out, None, src_v, dst_v,
              rows_a, rows_b, table_sh, acc_sh, g_a, g_b, s_a, s_b,
              None, None)

    return seg


_seg_sum_deg = _make_seg_sum(True)
_seg_sum = _make_seg_sum(False)


def _elu(v):
    return jnp.where(v > 0, v, jnp.exp(v) - 1.0)


def _row_spec(width):
    return pl.BlockSpec((_BLK, width), lambda i: (i, 0))


def _full_spec(shape):
    nd = len(shape)
    return pl.BlockSpec(shape, lambda i: (0,) * nd)


def _acc_spec(width):
    return pl.BlockSpec((_NC, _BLK, width), lambda i: (0, i, 0))


def _dense0_body(t_ref, x_ref, q_ref, fr_ref, wt1, bt1, wt2, bt2,
                 wl0x, wl0q, wr0x, wr0q, temb_ref, p0_ref, r0_ref):
    tt = t_ref[...]
    ph = tt * fr_ref[...]
    lane = lax.broadcasted_iota(jnp.int32, ph.shape, 1)
    temb0 = jnp.where(lane < _NHID // 2, jnp.sin(ph), jnp.cos(ph))
    hmid = _elu(jnp.dot(temb0, wt1[...], preferred_element_type=jnp.float32)
                + bt1[...])
    temb = jnp.dot(hmid, wt2[...], preferred_element_type=jnp.float32) + bt2[...]
    temb_ref[...] = temb
    x = x_ref[...]
    q = q_ref[...]
    p0_ref[...] = (jnp.dot(x, wl0x[...], preferred_element_type=jnp.float32)
                   + jnp.dot(q, wl0q[...], preferred_element_type=jnp.float32))
    r0_ref[...] = (jnp.dot(x, wr0x[...], preferred_element_type=jnp.float32)
                   + jnp.dot(q, wr0q[...], preferred_element_type=jnp.float32))


def _dense0(t2, x, q, fr, wt1, bt1, wt2, bt2, wl0x, wl0q, wr0x, wr0q):
    grid = (_N // _BLK,)
    return pl.pallas_call(
        _dense0_body,
        grid=grid,
        in_specs=[
            _row_spec(1), _row_spec(_NFEAT), _row_spec(_NLABEL),
            _full_spec(fr.shape), _full_spec(wt1.shape), _full_spec(bt1.shape),
            _full_spec(wt2.shape), _full_spec(bt2.shape),
            _full_spec(wl0x.shape), _full_spec(wl0q.shape),
            _full_spec(wr0x.shape), _full_spec(wr0q.shape),
        ],
        out_specs=[_row_spec(_NHID), _row_spec(_NHID), _row_spec(_NHID)],
        out_shape=[jax.ShapeDtypeStruct((_N, _NHID), jnp.float32)] * 3,
    )(t2, x, q, fr, wt1, bt1, wt2, bt2, wl0x, wl0q, wr0x, wr0q)


def _sage_post(acc, dg, r, bl, temb):
    agg = acc[0] + acc[1]
    deg = dg[0, :, 0:1] + dg[1, :, 0:1]
    out = agg / jnp.maximum(deg, 1.0) + bl[...] + r[...]
    nrm = jnp.sqrt(jnp.sum(out * out, axis=1, keepdims=True))
    out = out / jnp.maximum(nrm, 1e-12)
    return jnp.maximum(out + temb[...], 0.0)


def _dense1_body(acc, dg, r0, temb, q_ref, bl0,
                 wl1h, wl1q, wr1h, wr1q, p1_ref, r1_ref):
    h = _sage_post(acc, dg, r0, bl0, temb)
    q = q_ref[...]
    p1_ref[...] = (jnp.dot(h, wl1h[...], preferred_element_type=jnp.float32)
                   + jnp.dot(q, wl1q[...], preferred_element_type=jnp.float32))
    r1_ref[...] = (jnp.dot(h, wr1h[...], preferred_element_type=jnp.float32)
                   + jnp.dot(q, wr1q[...], preferred_element_type=jnp.float32))


def _dense1(acc, dg, r0, temb, q, bl0, wl1h, wl1q, wr1h, wr1q):
    grid = (_N // _BLK,)
    return pl.pallas_call(
        _dense1_body,
        grid=grid,
        in_specs=[
            _acc_spec(_NHID), _acc_spec(_DEGW),
            _row_spec(_NHID), _row_spec(_NHID), _row_spec(_NLABEL),
            _full_spec(bl0.shape),
            _full_spec(wl1h.shape), _full_spec(wl1q.shape),
            _full_spec(wr1h.shape), _full_spec(wr1q.shape),
        ],
        out_specs=[_row_spec(_NHID), _row_spec(_NHID)],
        out_shape=[jax.ShapeDtypeStruct((_N, _NHID), jnp.float32)] * 2,
    )(acc, dg, r0, temb, q, bl0, wl1h, wl1q, wr1h, wr1q)


def _dense2_body(acc, dg, r1, temb, q_ref, bl1,
                 wf1h, wf1q, bf1, wf2, bf2, out_ref):
    h = _sage_post(acc, dg, r1, bl1, temb)
    q = q_ref[...]
    f = _elu(jnp.dot(h, wf1h[...], preferred_element_type=jnp.float32)
             + jnp.dot(q, wf1q[...], preferred_element_type=jnp.float32)
             + bf1[...])
    out_ref[...] = jnp.dot(f, wf2[...], preferred_element_type=jnp.float32) + bf2[...]


def _dense2(acc, dg, r1, temb, q, bl1, wf1h, wf1q, bf1, wf2, bf2):
    grid = (_N // _BLK,)
    return pl.pallas_call(
        _dense2_body,
        grid=grid,
        in_specs=[
            _acc_spec(_NHID), _acc_spec(_DEGW),
            _row_spec(_NHID), _row_spec(_NHID), _row_spec(_NLABEL),
            _full_spec(bl1.shape),
            _full_spec(wf1h.shape), _full_spec(wf1q.shape),
            _full_spec(bf1.shape), _full_spec(wf2.shape), _full_spec(bf2.shape),
        ],
        out_specs=[_row_spec(_NLABEL)],
        out_shape=[jax.ShapeDtypeStruct((_N, _NLABEL), jnp.float32)],
    )(acc, dg, r1, temb, q, bl1, wf1h, wf1q, bf1, wf2, bf2)[0]


_FREQS = np.exp(np.arange(_NHID // 2, dtype=np.float64)
                * (-math.log(10000.0) / (_NHID // 2 - 1))).astype(np.float32)


def kernel(x, q_Y_sample, adj, t, num_steps, W_t1, b_t1, W_t2, b_t2,
           Wl0, bl0, Wr0, Wl1, bl1, Wr1, Wf1, bf1, Wf2, bf2):
    q = q_Y_sample
    adj32 = adj.astype(jnp.int32)

    t2 = (t / num_steps * num_steps * 4.0).reshape(_N, 1)
    fr = jnp.asarray(np.concatenate([_FREQS, _FREQS])[None, :])

    temb, p0, r0 = _dense0(
        t2, x, q, fr, W_t1, b_t1.reshape(1, -1), W_t2, b_t2.reshape(1, -1),
        Wl0[:_NFEAT], Wl0[_NFEAT:], Wr0[:_NFEAT], Wr0[_NFEAT:])

    acc0, deg0 = _seg_sum_deg(p0, adj32)

    p1, r1 = _dense1(
        acc0, deg0, r0, temb, q, bl0.reshape(1, -1),
        Wl1[:_NHID], Wl1[_NHID:], Wr1[:_NHID], Wr1[_NHID:])

    acc1 = _seg_sum(p1, adj32)

    out = _dense2(
        acc1, deg0, r1, temb, q, bl1.reshape(1, -1),
        Wf1[:_NHID], Wf1[_NHID:], bf1.reshape(1, -1), Wf2, bf2.reshape(1, -1))
    return out


# speedup vs baseline: 1.0574x; 1.0258x over previous
import functools
import math

import jax
import jax.numpy as jnp
import numpy as np
from jax import lax
from jax.experimental import pallas as pl
from jax.experimental.pallas import tpu as pltpu
from jax.experimental.pallas import tpu_sc as plsc

_N = 10000
_E = 320000
_NFEAT = 128
_NLABEL = 10
_NHID = 64

_NC = 2
_NS = 16
_W = _NC * _NS
_CHUNK = 128
_EW = _E // _W
_NT = _EW // _CHUNK
_TAIL = _EW - _NT * _CHUNK
_DEGW = 16

_BLK = 2000


def _make_seg_sum(with_deg):
    mesh = plsc.VectorSubcoreMesh(core_axis_name="c", subcore_axis_name="s")
    out_type = [jax.ShapeDtypeStruct((_NC, _N, _NHID), jnp.float32)]
    scratch = [
        pltpu.VMEM((_EW,), jnp.int32),
        pltpu.VMEM((_EW,), jnp.int32),
        pltpu.VMEM((_CHUNK, _NHID), jnp.float32),
        pltpu.VMEM((_CHUNK, _NHID), jnp.float32),
        pltpu.VMEM_SHARED((_N, _NHID), jnp.float32),
        pltpu.VMEM_SHARED((_N, _NHID), jnp.float32),
        pltpu.SemaphoreType.DMA,
        pltpu.SemaphoreType.DMA,
        pltpu.SemaphoreType.DMA,
        pltpu.SemaphoreType.DMA,
    ]
    if with_deg:
        out_type.append(jax.ShapeDtypeStruct((_NC, _N, _DEGW), jnp.float32))
        scratch += [
            pltpu.VMEM((_CHUNK, _DEGW), jnp.float32),
            pltpu.VMEM_SHARED((_N, _DEGW), jnp.float32),
        ]

    params = pltpu.CompilerParams(use_tc_tiling_on_sc=False)

    _ZCH = ((0, 128), (128, 128), (256, 128), (384, 128), (512, 113))

    def build(table, adj, out, degout, src_v, dst_v,
              rows_a, rows_b, table_sh, acc_sh, g_a, g_b, s_a, s_b,
              ones_v, deg_sh):
        s = lax.axis_index("s")
        c = lax.axis_index("c")
        wid = s * _NC + c
        zr = _N // _NS
        cp_t = pltpu.async_copy(table.at[pl.ds(s * zr, zr)],
                                table_sh.at[pl.ds(s * zr, zr)], g_a)
        cp_s = pltpu.async_copy(adj.at[0, pl.ds(wid * _EW, _EW)], src_v, g_b)
        cp_d = pltpu.async_copy(adj.at[1, pl.ds(wid * _EW, _EW)], dst_v, s_a)
        zv = jnp.zeros((16,), jnp.float32)
        for r in range(_CHUNK):
            for cb in range(_NHID // 16):
                rows_a[r, pl.ds(cb * 16, 16)] = zv
        if with_deg:
            for r in range(_CHUNK):
                ones_v[r, pl.ds(0, _DEGW)] = zv
        base = s * zr
        for kk, sz in _ZCH:
            pltpu.sync_copy(rows_a.at[pl.ds(0, sz)],
                            acc_sh.at[pl.ds(base + kk, sz)])
        if with_deg:
            for kk, sz in _ZCH:
                pltpu.sync_copy(ones_v.at[pl.ds(0, sz)],
                                deg_sh.at[pl.ds(base + kk, sz)])
            ov = jnp.ones((16,), jnp.float32)
            for r in range(_CHUNK):
                ones_v[r, pl.ds(0, _DEGW)] = ov
        cp_t.wait()
        cp_s.wait()
        cp_d.wait()
        plsc.subcore_barrier()

        def fire_gather(buf, i, sem):
            pltpu.async_copy(
                table_sh.at[src_v.at[pl.ds(i * _CHUNK, _CHUNK)]], buf, sem)

        def wait_gather(buf, sem):
            pltpu.make_async_copy(table.at[pl.ds(0, _CHUNK)], buf, sem).wait()

        def fire_scatter(buf, i, sem):
            idx = dst_v.at[pl.ds(i * _CHUNK, _CHUNK)]
            pltpu.async_copy(buf, acc_sh.at[idx], sem, add=True)
            if with_deg:
                pltpu.async_copy(ones_v, deg_sh.at[idx], sem, add=True)

        def wait_scatter(buf, sem):
            pltpu.make_async_copy(buf, acc_sh.at[pl.ds(0, _CHUNK)], sem).wait()
            if with_deg:
                pltpu.make_async_copy(ones_v, deg_sh.at[pl.ds(0, _CHUNK)],
                                      sem).wait()

        fire_gather(rows_a, 0, g_a)

        def body2(k, carry):
            i0 = k * 2
            for (off, cur, curg, curs, oth, othg, oths) in (
                    (0, rows_a, g_a, s_a, rows_b, g_b, s_b),
                    (1, rows_b, g_b, s_b, rows_a, g_a, s_a)):
                i = i0 + off
                wait_gather(cur, curg)
                fire_scatter(cur, i, curs)

                @pl.when(i >= 1)
                def _():
                    wait_scatter(oth, oths)

                @pl.when(i + 1 < _NT)
                def _():
                    fire_gather(oth, i + 1, othg)
            return carry

        lax.fori_loop(0, _NT // 2, body2, 0)
        wait_scatter(rows_b, s_b)

        tb = _NT * _CHUNK
        pltpu.async_copy(table_sh.at[src_v.at[pl.ds(tb, _TAIL)]],
                         rows_a.at[pl.ds(0, _TAIL)], g_a).wait()
        pltpu.sync_copy(rows_a.at[pl.ds(0, _TAIL)],
                        acc_sh.at[dst_v.at[pl.ds(tb, _TAIL)]], add=True)
        if with_deg:
            pltpu.sync_copy(ones_v.at[pl.ds(0, _TAIL)],
                            deg_sh.at[dst_v.at[pl.ds(tb, _TAIL)]], add=True)

        plsc.subcore_barrier()
        orow = _N // _NS
        pltpu.sync_copy(acc_sh.at[pl.ds(s * orow, orow)],
                        out.at[c, pl.ds(s * orow, orow)])
        if with_deg:
            pltpu.sync_copy(deg_sh.at[pl.ds(s * orow, orow)],
                            degout.at[c, pl.ds(s * orow, orow)])

    if with_deg:
        @functools.partial(pl.kernel, mesh=mesh, out_type=out_type,
                           scratch_types=scratch, compiler_params=params)
        def seg(table, adj, out, degout, src_v, dst_v,
                rows_a, rows_b, table_sh, acc_sh, g_a, g_b, s_a, s_b,
                ones_v, deg_sh):
            build(table, adj, out, degout, src_v, dst_v,
                  rows_a, rows_b, table_sh, acc_sh, g_a, g_b, s_a, s_b,
                  ones_v, deg_sh)

        return seg

    @functools.partial(pl.kernel, mesh=mesh, out_type=out_type[0],
                       scratch_types=scratch, compiler_params=params)
    def seg(table, adj, out, src_v, dst_v,
            rows_a, rows_b, table_sh, acc_sh, g_a, g_b, s_a, s_b):
        build(table, adj, out, None, src_v, dst_v,
              rows_a, rows_b, table_sh, acc_sh, g_a, g_b, s_a, s_b,
              None, None)

    return seg


_seg_sum_deg = _make_seg_sum(True)
_seg_sum = _make_seg_sum(False)


def _elu(v):
    return jnp.where(v > 0, v, jnp.exp(v) - 1.0)


def _row_spec(width):
    return pl.BlockSpec((_BLK, width), lambda i: (i, 0))


def _full_spec(shape):
    nd = len(shape)
    return pl.BlockSpec(shape, lambda i: (0,) * nd)


def _acc_spec(width):
    return pl.BlockSpec((_NC, _BLK, width), lambda i: (0, i, 0))


def _dense0_body(t_ref, x_ref, q_ref, fr_ref, wt1, bt1, wt2, bt2,
                 wl0x, wl0q, wr0x, wr0q, temb_ref, p0_ref, r0_ref):
    tt = t_ref[...]
    ph = tt * fr_ref[...]
    lane = lax.broadcasted_iota(jnp.int32, ph.shape, 1)
    temb0 = jnp.where(lane < _NHID // 2, jnp.sin(ph), jnp.cos(ph))
    hmid = _elu(jnp.dot(temb0, wt1[...], preferred_element_type=jnp.float32)
                + bt1[...])
    temb = jnp.dot(hmid, wt2[...], preferred_element_type=jnp.float32) + bt2[...]
    temb_ref[...] = temb
    x = x_ref[...]
    q = q_ref[...]
    p0_ref[...] = (jnp.dot(x, wl0x[...], preferred_element_type=jnp.float32)
                   + jnp.dot(q, wl0q[...], preferred_element_type=jnp.float32))
    r0_ref[...] = (jnp.dot(x, wr0x[...], preferred_element_type=jnp.float32)
                   + jnp.dot(q, wr0q[...], preferred_element_type=jnp.float32))


def _dense0(t2, x, q, fr, wt1, bt1, wt2, bt2, wl0x, wl0q, wr0x, wr0q):
    grid = (_N // _BLK,)
    return pl.pallas_call(
        _dense0_body,
        grid=grid,
        in_specs=[
            _row_spec(1), _row_spec(_NFEAT), _row_spec(_NLABEL),
            _full_spec(fr.shape), _full_spec(wt1.shape), _full_spec(bt1.shape),
            _full_spec(wt2.shape), _full_spec(bt2.shape),
            _full_spec(wl0x.shape), _full_spec(wl0q.shape),
            _full_spec(wr0x.shape), _full_spec(wr0q.shape),
        ],
        out_specs=[_row_spec(_NHID), _row_spec(_NHID), _row_spec(_NHID)],
        out_shape=[jax.ShapeDtypeStruct((_N, _NHID), jnp.float32)] * 3,
    )(t2, x, q, fr, wt1, bt1, wt2, bt2, wl0x, wl0q, wr0x, wr0q)


def _sage_post(acc, dg, r, bl, temb):
    agg = acc[0] + acc[1]
    deg = dg[0, :, 0:1] + dg[1, :, 0:1]
    out = agg / jnp.maximum(deg, 1.0) + bl[...] + r[...]
    nrm = jnp.sqrt(jnp.sum(out * out, axis=1, keepdims=True))
    out = out / jnp.maximum(nrm, 1e-12)
    return jnp.maximum(out + temb[...], 0.0)


def _dense1_body(acc, dg, r0, temb, q_ref, bl0,
                 wl1h, wl1q, wr1h, wr1q, p1_ref, r1_ref):
    h = _sage_post(acc, dg, r0, bl0, temb)
    q = q_ref[...]
    p1_ref[...] = (jnp.dot(h, wl1h[...], preferred_element_type=jnp.float32)
                   + jnp.dot(q, wl1q[...], preferred_element_type=jnp.float32))
    r1_ref[...] = (jnp.dot(h, wr1h[...], preferred_element_type=jnp.float32)
                   + jnp.dot(q, wr1q[...], preferred_element_type=jnp.float32))


def _dense1(acc, dg, r0, temb, q, bl0, wl1h, wl1q, wr1h, wr1q):
    grid = (_N // _BLK,)
    return pl.pallas_call(
        _dense1_body,
        grid=grid,
        in_specs=[
            _acc_spec(_NHID), _acc_spec(_DEGW),
            _row_spec(_NHID), _row_spec(_NHID), _row_spec(_NLABEL),
            _full_spec(bl0.shape),
            _full_spec(wl1h.shape), _full_spec(wl1q.shape),
            _full_spec(wr1h.shape), _full_spec(wr1q.shape),
        ],
        out_specs=[_row_spec(_NHID), _row_spec(_NHID)],
        out_shape=[jax.ShapeDtypeStruct((_N, _NHID), jnp.float32)] * 2,
    )(acc, dg, r0, temb, q, bl0, wl1h, wl1q, wr1h, wr1q)


def _dense2_body(acc, dg, r1, temb, q_ref, bl1,
                 wf1h, wf1q, bf1, wf2, bf2, out_ref):
    h = _sage_post(acc, dg, r1, bl1, temb)
    q = q_ref[...]
    f = _elu(jnp.dot(h, wf1h[...], preferred_element_type=jnp.float32)
             + jnp.dot(q, wf1q[...], preferred_element_type=jnp.float32)
             + bf1[...])
    out_ref[...] = jnp.dot(f, wf2[...], preferred_element_type=jnp.float32) + bf2[...]


def _dense2(acc, dg, r1, temb, q, bl1, wf1h, wf1q, bf1, wf2, bf2):
    grid = (_N // _BLK,)
    return pl.pallas_call(
        _dense2_body,
        grid=grid,
        in_specs=[
            _acc_spec(_NHID), _acc_spec(_DEGW),
            _row_spec(_NHID), _row_spec(_NHID), _row_spec(_NLABEL),
            _full_spec(bl1.shape),
            _full_spec(wf1h.shape), _full_spec(wf1q.shape),
            _full_spec(bf1.shape), _full_spec(wf2.shape), _full_spec(bf2.shape),
        ],
        out_specs=[_row_spec(_NLABEL)],
        out_shape=[jax.ShapeDtypeStruct((_N, _NLABEL), jnp.float32)],
    )(acc, dg, r1, temb, q, bl1, wf1h, wf1q, bf1, wf2, bf2)[0]


_FREQS = np.exp(np.arange(_NHID // 2, dtype=np.float64)
                * (-math.log(10000.0) / (_NHID // 2 - 1))).astype(np.float32)


def kernel(x, q_Y_sample, adj, t, num_steps, W_t1, b_t1, W_t2, b_t2,
           Wl0, bl0, Wr0, Wl1, bl1, Wr1, Wf1, bf1, Wf2, bf2):
    q = q_Y_sample
    adj32 = adj.astype(jnp.int32)

    t2 = (t / num_steps * num_steps * 4.0).reshape(_N, 1)
    fr = jnp.asarray(np.concatenate([_FREQS, _FREQS])[None, :])

    temb, p0, r0 = _dense0(
        t2, x, q, fr, W_t1, b_t1.reshape(1, -1), W_t2, b_t2.reshape(1, -1),
        Wl0[:_NFEAT], Wl0[_NFEAT:], Wr0[:_NFEAT], Wr0[_NFEAT:])

    acc0, deg0 = _seg_sum_deg(p0, adj32)

    p1, r1 = _dense1(
        acc0, deg0, r0, temb, q, bl0.reshape(1, -1),
        Wl1[:_NHID], Wl1[_NHID:], Wr1[:_NHID], Wr1[_NHID:])

    acc1 = _seg_sum(p1, adj32)

    out = _dense2(
        acc1, deg0, r1, temb, q, bl1.reshape(1, -1),
        Wf1[:_NHID], Wf1[_NHID:], bf1.reshape(1, -1), Wf2, bf2.reshape(1, -1))
    return out


# speedup vs baseline: 1.0689x; 1.0108x over previous
import functools
import math

import jax
import jax.numpy as jnp
import numpy as np
from jax import lax
from jax.experimental import pallas as pl
from jax.experimental.pallas import tpu as pltpu
from jax.experimental.pallas import tpu_sc as plsc

_N = 10000
_E = 320000
_NFEAT = 128
_NLABEL = 10
_NHID = 64

_NC = 2
_NS = 16
_W = _NC * _NS
_CHUNK = 128
_EW = _E // _W
_NT = _EW // _CHUNK
_TAIL = _EW - _NT * _CHUNK
_DEGW = 16

_BLK = 2000


def _make_seg_sum(with_deg):
    mesh = plsc.VectorSubcoreMesh(core_axis_name="c", subcore_axis_name="s")
    out_type = [jax.ShapeDtypeStruct((_NC, _N, _NHID), jnp.float32)]
    scratch = [
        pltpu.VMEM((_EW,), jnp.int32),
        pltpu.VMEM((_EW,), jnp.int32),
        pltpu.VMEM((_CHUNK, _NHID), jnp.float32),
        pltpu.VMEM((_CHUNK, _NHID), jnp.float32),
        pltpu.VMEM_SHARED((_N, _NHID), jnp.float32),
        pltpu.VMEM_SHARED((_N, _NHID), jnp.float32),
        pltpu.SemaphoreType.DMA,
        pltpu.SemaphoreType.DMA,
        pltpu.SemaphoreType.DMA,
        pltpu.SemaphoreType.DMA,
    ]
    if with_deg:
        out_type.append(jax.ShapeDtypeStruct((_NC, _N, _DEGW), jnp.float32))
        scratch += [
            pltpu.VMEM((_CHUNK, _DEGW), jnp.float32),
            pltpu.VMEM_SHARED((_N, _DEGW), jnp.float32),
        ]

    params = pltpu.CompilerParams(use_tc_tiling_on_sc=False)

    _ZCH = ((0, 128), (128, 128), (256, 128), (384, 128), (512, 113))

    def build(table, adj, out, degout, src_v, dst_v,
              rows_a, rows_b, table_sh, acc_sh, g_a, g_b, s_a, s_b,
              ones_v, deg_sh):
        s = lax.axis_index("s")
        c = lax.axis_index("c")
        wid = s * _NC + c
        zr = _N // _NS
        cp_t = pltpu.async_copy(table.at[pl.ds(s * zr, zr)],
                                table_sh.at[pl.ds(s * zr, zr)], g_a)
        cp_s = pltpu.async_copy(adj.at[0, pl.ds(wid * _EW, _EW)], src_v, g_b)
        cp_d = pltpu.async_copy(adj.at[1, pl.ds(wid * _EW, _EW)], dst_v, s_a)
        zv = jnp.zeros((16,), jnp.float32)
        for r in range(_CHUNK):
            for cb in range(_NHID // 16):
                rows_a[r, pl.ds(cb * 16, 16)] = zv
        if with_deg:
            for r in range(_CHUNK):
                ones_v[r, pl.ds(0, _DEGW)] = zv
        base = s * zr
        for kk, sz in _ZCH:
            pltpu.sync_copy(rows_a.at[pl.ds(0, sz)],
                            acc_sh.at[pl.ds(base + kk, sz)])
        if with_deg:
            for kk, sz in _ZCH:
                pltpu.sync_copy(ones_v.at[pl.ds(0, sz)],
                                deg_sh.at[pl.ds(base + kk, sz)])
            ov = jnp.ones((16,), jnp.float32)
            for r in range(_CHUNK):
                ones_v[r, pl.ds(0, _DEGW)] = ov
        cp_t.wait()
        cp_s.wait()
        cp_d.wait()
        plsc.subcore_barrier()

        def fire_gather(buf, i, sem):
            pltpu.async_copy(
                table_sh.at[src_v.at[pl.ds(i * _CHUNK, _CHUNK)]], buf, sem)

        def wait_gather(buf, sem):
            pltpu.make_async_copy(table.at[pl.ds(0, _CHUNK)], buf, sem).wait()

        def fire_scatter(buf, i, sem):
            idx = dst_v.at[pl.ds(i * _CHUNK, _CHUNK)]
            pltpu.async_copy(buf, acc_sh.at[idx], sem, add=True)
            if with_deg:
                pltpu.async_copy(ones_v, deg_sh.at[idx], sem, add=True)

        def wait_scatter(buf, sem):
            pltpu.make_async_copy(buf, acc_sh.at[pl.ds(0, _CHUNK)], sem).wait()
            if with_deg:
                pltpu.make_async_copy(ones_v, deg_sh.at[pl.ds(0, _CHUNK)],
                                      sem).wait()

        fire_gather(rows_a, 0, g_a)

        def body2(k, carry):
            i0 = k * 2
            for (off, cur, curg, curs, oth, othg, oths) in (
                    (0, rows_a, g_a, s_a, rows_b, g_b, s_b),
                    (1, rows_b, g_b, s_b, rows_a, g_a, s_a)):
                i = i0 + off
                wait_gather(cur, curg)
                fire_scatter(cur, i, curs)

                @pl.when(i >= 1)
                def _():
                    wait_scatter(oth, oths)

                @pl.when(i + 1 < _NT)
                def _():
                    fire_gather(oth, i + 1, othg)
            return carry

        lax.fori_loop(0, _NT // 2, body2, 0)
        wait_scatter(rows_b, s_b)

        tb = _NT * _CHUNK
        pltpu.async_copy(table_sh.at[src_v.at[pl.ds(tb, _TAIL)]],
                         rows_a.at[pl.ds(0, _TAIL)], g_a).wait()
        pltpu.sync_copy(rows_a.at[pl.ds(0, _TAIL)],
                        acc_sh.at[dst_v.at[pl.ds(tb, _TAIL)]], add=True)
        if with_deg:
            pltpu.sync_copy(ones_v.at[pl.ds(0, _TAIL)],
                            deg_sh.at[dst_v.at[pl.ds(tb, _TAIL)]], add=True)

        plsc.subcore_barrier()
        orow = _N // _NS
        pltpu.sync_copy(acc_sh.at[pl.ds(s * orow, orow)],
                        out.at[c, pl.ds(s * orow, orow)])
        if with_deg:
            pltpu.sync_copy(deg_sh.at[pl.ds(s * orow, orow)],
                            degout.at[c, pl.ds(s * orow, orow)])

    if with_deg:
        @functools.partial(pl.kernel, mesh=mesh, out_type=out_type,
                           scratch_types=scratch, compiler_params=params)
        def seg(table, adj, out, degout, src_v, dst_v,
                rows_a, rows_b, table_sh, acc_sh, g_a, g_b, s_a, s_b,
                ones_v, deg_sh):
            build(table, adj, out, degout, src_v, dst_v,
                  rows_a, rows_b, table_sh, acc_sh, g_a, g_b, s_a, s_b,
                  ones_v, deg_sh)

        return seg

    @functools.partial(pl.kernel, mesh=mesh, out_type=out_type[0],
                       scratch_types=scratch, compiler_params=params)
    def seg(table, adj, out, src_v, dst_v,
            rows_a, rows_b, table_sh, acc_sh, g_a, g_b, s_a, s_b):
        build(table, adj, out, None, src_v, dst_v,
              rows_a, rows_b, table_sh, acc_sh, g_a, g_b, s_a, s_b,
              None, None)

    return seg


_seg_sum_deg = _make_seg_sum(True)
_seg_sum = _make_seg_sum(False)


def _elu(v):
    return jnp.where(v > 0, v, jnp.exp(v) - 1.0)


def _row_spec(width):
    return pl.BlockSpec((_BLK, width), lambda i: (i, 0))


def _full_spec(shape):
    nd = len(shape)
    return pl.BlockSpec(shape, lambda i: (0,) * nd)


def _acc_spec(width):
    return pl.BlockSpec((_NC, _BLK, width), lambda i: (0, i, 0))


def _dense0_body(t_ref, x_ref, q_ref, fr_ref, wt1, bt1, wt2, bt2,
                 wl0x, wl0q, temb_ref, p0_ref, r0_ref):
    tt = t_ref[...]
    ph = tt * fr_ref[...]
    lane = lax.broadcasted_iota(jnp.int32, ph.shape, 1)
    temb0 = jnp.where(lane < _NHID // 2, jnp.sin(ph), jnp.cos(ph))
    hmid = _elu(jnp.dot(temb0, wt1[...], preferred_element_type=jnp.float32)
                + bt1[...])
    temb = jnp.dot(hmid, wt2[...], preferred_element_type=jnp.float32) + bt2[...]
    temb_ref[...] = temb
    x = x_ref[...]
    q = q_ref[...]
    big = (jnp.dot(x, wl0x[...], preferred_element_type=jnp.float32)
           + jnp.dot(q, wl0q[...], preferred_element_type=jnp.float32))
    p0_ref[...] = big[:, :_NHID]
    r0_ref[...] = big[:, _NHID:]


def _dense0(t2, x, q, fr, wt1, bt1, wt2, bt2, wl0x, wl0q):
    grid = (_N // _BLK,)
    return pl.pallas_call(
        _dense0_body,
        grid=grid,
        in_specs=[
            _row_spec(1), _row_spec(_NFEAT), _row_spec(_NLABEL),
            _full_spec(fr.shape), _full_spec(wt1.shape), _full_spec(bt1.shape),
            _full_spec(wt2.shape), _full_spec(bt2.shape),
            _full_spec(wl0x.shape), _full_spec(wl0q.shape),
        ],
        out_specs=[_row_spec(_NHID), _row_spec(_NHID), _row_spec(_NHID)],
        out_shape=[jax.ShapeDtypeStruct((_N, _NHID), jnp.float32)] * 3,
    )(t2, x, q, fr, wt1, bt1, wt2, bt2, wl0x, wl0q)


def _sage_post(acc, dg, r, bl, temb):
    agg = acc[0] + acc[1]
    deg = dg[0, :, 0:1] + dg[1, :, 0:1]
    out = agg / jnp.maximum(deg, 1.0) + bl[...] + r[...]
    nrm = jnp.sqrt(jnp.sum(out * out, axis=1, keepdims=True))
    out = out / jnp.maximum(nrm, 1e-12)
    return jnp.maximum(out + temb[...], 0.0)


def _dense1_body(acc, dg, r0, temb, q_ref, bl0,
                 wl1h, wl1q, wr1h, wr1q, p1_ref, r1_ref):
    h = _sage_post(acc, dg, r0, bl0, temb)
    q = q_ref[...]
    p1_ref[...] = (jnp.dot(h, wl1h[...], preferred_element_type=jnp.float32)
                   + jnp.dot(q, wl1q[...], preferred_element_type=jnp.float32))
    r1_ref[...] = (jnp.dot(h, wr1h[...], preferred_element_type=jnp.float32)
                   + jnp.dot(q, wr1q[...], preferred_element_type=jnp.float32))


def _dense1(acc, dg, r0, temb, q, bl0, wl1h, wl1q, wr1h, wr1q):
    grid = (_N // _BLK,)
    return pl.pallas_call(
        _dense1_body,
        grid=grid,
        in_specs=[
            _acc_spec(_NHID), _acc_spec(_DEGW),
            _row_spec(_NHID), _row_spec(_NHID), _row_spec(_NLABEL),
            _full_spec(bl0.shape),
            _full_spec(wl1h.shape), _full_spec(wl1q.shape),
            _full_spec(wr1h.shape), _full_spec(wr1q.shape),
        ],
        out_specs=[_row_spec(_NHID), _row_spec(_NHID)],
        out_shape=[jax.ShapeDtypeStruct((_N, _NHID), jnp.float32)] * 2,
    )(acc, dg, r0, temb, q, bl0, wl1h, wl1q, wr1h, wr1q)


def _dense2_body(acc, dg, r1, temb, q_ref, bl1,
                 wf1h, wf1q, bf1, wf2, bf2, out_ref):
    h = _sage_post(acc, dg, r1, bl1, temb)
    q = q_ref[...]
    f = _elu(jnp.dot(h, wf1h[...], preferred_element_type=jnp.float32)
             + jnp.dot(q, wf1q[...], preferred_element_type=jnp.float32)
             + bf1[...])
    out_ref[...] = jnp.dot(f, wf2[...], preferred_element_type=jnp.float32) + bf2[...]


def _dense2(acc, dg, r1, temb, q, bl1, wf1h, wf1q, bf1, wf2, bf2):
    grid = (_N // _BLK,)
    return pl.pallas_call(
        _dense2_body,
        grid=grid,
        in_specs=[
            _acc_spec(_NHID), _acc_spec(_DEGW),
            _row_spec(_NHID), _row_spec(_NHID), _row_spec(_NLABEL),
            _full_spec(bl1.shape),
            _full_spec(wf1h.shape), _full_spec(wf1q.shape),
            _full_spec(bf1.shape), _full_spec(wf2.shape), _full_spec(bf2.shape),
        ],
        out_specs=[_row_spec(_NLABEL)],
        out_shape=[jax.ShapeDtypeStruct((_N, _NLABEL), jnp.float32)],
    )(acc, dg, r1, temb, q, bl1, wf1h, wf1q, bf1, wf2, bf2)[0]


_FREQS = np.exp(np.arange(_NHID // 2, dtype=np.float64)
                * (-math.log(10000.0) / (_NHID // 2 - 1))).astype(np.float32)


def kernel(x, q_Y_sample, adj, t, num_steps, W_t1, b_t1, W_t2, b_t2,
           Wl0, bl0, Wr0, Wl1, bl1, Wr1, Wf1, bf1, Wf2, bf2):
    q = q_Y_sample
    adj32 = adj.astype(jnp.int32)

    t2 = (t / num_steps * num_steps * 4.0).reshape(_N, 1)
    fr = jnp.asarray(np.concatenate([_FREQS, _FREQS])[None, :])

    wlr_x = jnp.concatenate([Wl0[:_NFEAT], Wr0[:_NFEAT]], axis=1)
    wlr_q = jnp.concatenate([Wl0[_NFEAT:], Wr0[_NFEAT:]], axis=1)
    temb, p0, r0 = _dense0(
        t2, x, q, fr, W_t1, b_t1.reshape(1, -1), W_t2, b_t2.reshape(1, -1),
        wlr_x, wlr_q)

    acc0, deg0 = _seg_sum_deg(p0, adj32)

    p1, r1 = _dense1(
        acc0, deg0, r0, temb, q, bl0.reshape(1, -1),
        Wl1[:_NHID], Wl1[_NHID:], Wr1[:_NHID], Wr1[_NHID:])

    acc1 = _seg_sum(p1, adj32)

    out = _dense2(
        acc1, deg0, r1, temb, q, bl1.reshape(1, -1),
        Wf1[:_NHID], Wf1[_NHID:], bf1.reshape(1, -1), Wf2, bf2.reshape(1, -1))
    return out
